# baseline (speedup 1.0000x reference)
import functools

import jax
import jax.numpy as jnp
from jax import lax
from jax.experimental import pallas as pl
from jax.experimental.pallas import tpu as pltpu
from jax.experimental.pallas import tpu_sc as plsc

N = 10000
D = 128
E = 320000

CHUNK = 48
NWORKERS = 32
CPT = 210
EPT = CPT * CHUNK
E_PAD = NWORKERS * EPT
TRIPLES = CPT // 3
ACC_ROWS = 10112
ROWS_PER_TILE = ACC_ROWS // 16


@functools.partial(
    pl.kernel,
    mesh=plsc.VectorSubcoreMesh(core_axis_name="c", subcore_axis_name="s"),
    out_type=jax.ShapeDtypeStruct((2, ACC_ROWS, D), jnp.float32),
    scratch_types=[
        pltpu.VMEM((EPT,), jnp.int32),
        pltpu.VMEM((EPT,), jnp.int32),
        pltpu.VMEM((EPT,), jnp.float32),
        pltpu.VMEM((CHUNK, D), jnp.float32),
        pltpu.VMEM((CHUNK, D), jnp.float32),
        pltpu.VMEM((CHUNK, D), jnp.float32),
        pltpu.VMEM_SHARED((ACC_ROWS, D), jnp.float32),
        pltpu.SemaphoreType.DMA,
        pltpu.SemaphoreType.DMA,
        pltpu.SemaphoreType.DMA,
        pltpu.SemaphoreType.DMA,
        pltpu.SemaphoreType.DMA,
        pltpu.SemaphoreType.DMA,
    ],
)
def _sc_aggregate(src_hbm, dst_hbm, w_hbm, ego_hbm, zeros_hbm, out_hbm,
                  src_v, dst_v, w_v, r0, r1, r2, acc,
                  g0, g1, g2, s0, s1, s2):
    c = lax.axis_index("c")
    s = lax.axis_index("s")
    wid = s * 2 + c
    rows = (r0, r1, r2)
    gsems = (g0, g1, g2)
    ssems = (s0, s1, s2)

    pltpu.sync_copy(src_hbm.at[pl.ds(wid * EPT, EPT)], src_v)
    pltpu.sync_copy(dst_hbm.at[pl.ds(wid * EPT, EPT)], dst_v)
    pltpu.sync_copy(w_hbm.at[pl.ds(wid * EPT, EPT)], w_v)
    row0 = s * ROWS_PER_TILE
    pltpu.sync_copy(zeros_hbm, acc.at[pl.ds(row0, ROWS_PER_TILE)])

    def fire_g(b, k):
        pltpu.make_async_copy(ego_hbm.at[src_v.at[pl.ds(k * CHUNK, CHUNK)]],
                              rows[b], gsems[b]).start()

    def wait_g(b):
        pltpu.make_async_copy(ego_hbm.at[src_v.at[pl.ds(0, CHUNK)]],
                              rows[b], gsems[b]).wait()

    def fire_s(b, k):
        pltpu.make_async_copy(rows[b], acc.at[dst_v.at[pl.ds(k * CHUNK, CHUNK)]],
                              ssems[b]).start(add=True)

    def wait_s(b):
        pltpu.make_async_copy(rows[b], acc.at[dst_v.at[pl.ds(0, CHUNK)]],
                              ssems[b]).wait()

    def scale(b, k):
        def group_body(g, carry):
            w16 = w_v[pl.ds(k * CHUNK + g * 16, 16)]
            for lane in range(16):
                e = g * 16 + lane
                w = jnp.full((16,), w16[lane])
                for j in range(D // 16):
                    sl = pl.ds(j * 16, 16)
                    rows[b][e, sl] = rows[b][e, sl] * w
            return carry

        lax.fori_loop(0, CHUNK // 16, group_body, 0)

    fire_g(0, 0)
    plsc.subcore_barrier()

    def triple_body(t, carry):
        for u in range(3):
            j = t * 3 + u
            nb = (u + 1) % 3
            if u == 2:
                wait_s(nb)

                @pl.when(t < TRIPLES - 1)
                def _():
                    fire_g(nb, j + 1)
            else:

                @pl.when(t >= 1)
                def _():
                    wait_s(nb)

                fire_g(nb, j + 1)

            wait_g(u)
            scale(u, j)
            fire_s(u, j)
        return carry

    lax.fori_loop(0, TRIPLES, triple_body, 0)
    wait_s(1)
    wait_s(2)

    plsc.subcore_barrier()
    pltpu.sync_copy(acc.at[pl.ds(row0, ROWS_PER_TILE)],
                    out_hbm.at[c, pl.ds(row0, ROWS_PER_TILE)])


ROWS_BLK = 1000


def _tc_dense_body(ego_ref, p_ref, w1_ref, b1_ref, w2_ref, b2_ref, out_ref):
    side = p_ref[0] + p_ref[1]
    ego = ego_ref[...]
    dn = (((1,), (1,)), ((), ()))
    y1 = lax.dot_general(ego + side, w1_ref[...], dn,
                         preferred_element_type=jnp.float32) + b1_ref[...]
    y1 = jnp.where(y1 >= 0, y1, 0.01 * y1)
    y2 = lax.dot_general(ego * side, w2_ref[...], dn,
                         preferred_element_type=jnp.float32) + b2_ref[...]
    y2 = jnp.where(y2 >= 0, y2, 0.01 * y2)
    out_ref[...] = y1 + y2


_tc_dense = pl.pallas_call(
    _tc_dense_body,
    grid=(N // ROWS_BLK,),
    in_specs=[
        pl.BlockSpec((ROWS_BLK, D), lambda i: (i, 0)),
        pl.BlockSpec((2, ROWS_BLK, D), lambda i: (0, i, 0)),
        pl.BlockSpec((D, D), lambda i: (0, 0)),
        pl.BlockSpec((1, D), lambda i: (0, 0)),
        pl.BlockSpec((D, D), lambda i: (0, 0)),
        pl.BlockSpec((1, D), lambda i: (0, 0)),
    ],
    out_specs=pl.BlockSpec((ROWS_BLK, D), lambda i: (i, 0)),
    out_shape=jax.ShapeDtypeStruct((N, D), jnp.float32),
)


def kernel(ego_embeddings, edge_index, edge_weight, W1, b1, W2, b2):
    src = edge_index[0].astype(jnp.int32)
    dst = edge_index[1].astype(jnp.int32)
    pad = E_PAD - E
    srcp = jnp.concatenate([src, jnp.zeros((pad,), jnp.int32)])
    dstp = jnp.concatenate([dst, jnp.zeros((pad,), jnp.int32)])
    wp = jnp.concatenate([edge_weight, jnp.zeros((pad,), jnp.float32)])
    zeros = jnp.zeros((ROWS_PER_TILE, D), jnp.float32)
    partials = _sc_aggregate(srcp, dstp, wp, ego_embeddings, zeros)
    return _tc_dense(ego_embeddings, partials, W1, b1.reshape(1, D),
                     W2, b2.reshape(1, D))
